# Initial kernel scaffold; baseline (speedup 1.0000x reference)
import jax
import jax.numpy as jnp
from jax.experimental import pallas as pl


def kernel(x, edge_index, batch, Wn, bn, Wq, bq, Wk, bk, Wv, bv, Ws, bs):
    raise NotImplementedError("write your pallas kernel here")


# speedup vs baseline: 18.1234x; 18.1234x over previous
import functools

import jax
import jax.numpy as jnp
import numpy as np
from jax import lax
from jax.experimental import pallas as pl
from jax.experimental.pallas import tpu as pltpu
from jax.experimental.pallas import tpu_sc as plsc

H = 8
C = 64
L = 16
NW = 32


def _proj_body(x_ref, wn, bn, wqp, bqp, wkp, bkp, wv, bv, ws, bs,
               qt_ref, kt_ref, v_ref, skip_ref):
    xb = x_ref[...]
    h = jnp.dot(xb, wn[...], preferred_element_type=jnp.float32) + bn[...]
    qt_ref[...] = jnp.dot(h, wqp[...], preferred_element_type=jnp.float32) + bqp[...]
    kt_ref[...] = jnp.dot(h, wkp[...], preferred_element_type=jnp.float32) + bkp[...]
    v_ref[...] = jnp.dot(h, wv[...], preferred_element_type=jnp.float32) + bv[...]
    skip_ref[...] = jnp.dot(h, ws[...], preferred_element_type=jnp.float32) + bs[...]


def _run_proj(x, wn, bn, wqp, bqp, wkp, bkp, wv, bv, ws, bs):
    n, d = x.shape
    hc = wqp.shape[1]
    blk = 1000
    grid = n // blk
    full = lambda r, c: pl.BlockSpec((r, c), lambda i: (0, 0))
    return pl.pallas_call(
        _proj_body,
        grid=(grid,),
        in_specs=[
            pl.BlockSpec((blk, d), lambda i: (i, 0)),
            full(d, d), full(1, d),
            full(d, hc), full(1, hc),
            full(d, hc), full(1, hc),
            full(d, hc), full(1, hc),
            full(d, C), full(1, C),
        ],
        out_specs=[
            pl.BlockSpec((blk, hc), lambda i: (i, 0)),
            pl.BlockSpec((blk, hc), lambda i: (i, 0)),
            pl.BlockSpec((blk, hc), lambda i: (i, 0)),
            pl.BlockSpec((blk, C), lambda i: (i, 0)),
        ],
        out_shape=[
            jax.ShapeDtypeStruct((n, hc), jnp.float32),
            jax.ShapeDtypeStruct((n, hc), jnp.float32),
            jax.ShapeDtypeStruct((n, hc), jnp.float32),
            jax.ShapeDtypeStruct((n, C), jnp.float32),
        ],
    )(x, wn, bn, wqp, bqp, wkp, bkp, wv, bv, ws, bs)


def _make_kernel_a(n_pad, n_edges):
    ch = 40
    epw = n_edges // NW
    nch = epw // ch
    npt = n_pad // 16
    hc = H * C
    mesh = plsc.VectorSubcoreMesh(core_axis_name="c", subcore_axis_name="s")

    @functools.partial(
        pl.kernel,
        out_type=[
            jax.ShapeDtypeStruct((n_edges, L), jnp.float32),
            jax.ShapeDtypeStruct((n_pad, L), jnp.float32),
            jax.ShapeDtypeStruct((n_pad, L), jnp.float32),
        ],
        mesh=mesh,
        compiler_params=pltpu.CompilerParams(use_tc_tiling_on_sc=False),
        scratch_types=[
            pltpu.VMEM((ch,), jnp.int32),
            pltpu.VMEM((ch,), jnp.int32),
            pltpu.VMEM((ch, hc), jnp.float32),
            pltpu.VMEM((ch, hc), jnp.float32),
            pltpu.VMEM((ch, L), jnp.float32),
            pltpu.VMEM_SHARED((n_pad, L), jnp.float32),
            pltpu.SemaphoreType.DMA,
            pltpu.SemaphoreType.DMA,
        ],
    )
    def ka(qt, kt, srci, dsti, w_out, dp0, dp1,
           sbuf, dbuf, qbuf, kbuf, wbuf, denom_sp, sem_q, sem_k):
        cid = lax.axis_index("c")
        sid = lax.axis_index("s")
        wid = sid * 2 + cid

        zero16 = jnp.zeros((L,), jnp.float32)

        def zloop(i, carry):
            wbuf[i] = zero16
            return carry

        lax.fori_loop(0, ch, zloop, 0)

        def zcopy(i, carry):
            pltpu.sync_copy(wbuf, denom_sp.at[pl.ds(sid * npt + i * ch, ch)])
            return carry

        lax.fori_loop(0, npt // ch, zcopy, 0)
        plsc.subcore_barrier()

        ebase = wid * epw

        def chunk(g, carry):
            base = ebase + g * ch
            pltpu.sync_copy(srci.at[pl.ds(base, ch)], sbuf)
            pltpu.sync_copy(dsti.at[pl.ds(base, ch)], dbuf)
            cq = pltpu.async_copy(qt.at[dbuf], qbuf, sem_q)
            ck = pltpu.async_copy(kt.at[sbuf], kbuf, sem_k)
            cq.wait()
            ck.wait()

            def edge(e, ecarry):
                a0 = qbuf[e, pl.ds(0, L)] * kbuf[e, pl.ds(0, L)]
                a1 = qbuf[e, pl.ds(L, L)] * kbuf[e, pl.ds(L, L)]
                a2 = qbuf[e, pl.ds(2 * L, L)] * kbuf[e, pl.ds(2 * L, L)]
                a3 = qbuf[e, pl.ds(3 * L, L)] * kbuf[e, pl.ds(3 * L, L)]
                for i in range(4, 32, 4):
                    a0 = a0 + qbuf[e, pl.ds(i * L, L)] * kbuf[e, pl.ds(i * L, L)]
                    a1 = a1 + qbuf[e, pl.ds((i + 1) * L, L)] * kbuf[e, pl.ds((i + 1) * L, L)]
                    a2 = a2 + qbuf[e, pl.ds((i + 2) * L, L)] * kbuf[e, pl.ds((i + 2) * L, L)]
                    a3 = a3 + qbuf[e, pl.ds((i + 3) * L, L)] * kbuf[e, pl.ds((i + 3) * L, L)]
                acc = (a0 + a1) + (a2 + a3)
                logits = (acc + lax.rev(acc, (0,))) * 0.125
                wbuf[e] = jnp.exp(logits)
                return ecarry

            lax.fori_loop(0, ch, edge, 0)
            pltpu.sync_copy(wbuf, w_out.at[pl.ds(base, ch)])
            pltpu.sync_copy(wbuf, denom_sp.at[dbuf], add=True)
            return carry

        lax.fori_loop(0, nch, chunk, 0)
        plsc.subcore_barrier()

        nb = sid * npt

        @pl.when(cid == 0)
        def _():
            pltpu.sync_copy(denom_sp.at[pl.ds(nb, npt)], dp0.at[pl.ds(nb, npt)])

        @pl.when(cid == 1)
        def _():
            pltpu.sync_copy(denom_sp.at[pl.ds(nb, npt)], dp1.at[pl.ds(nb, npt)])

    return ka


def _make_kernel_b(n_pad, n_edges):
    ch = 40
    epw = n_edges // NW
    nch = epw // ch
    npt = n_pad // 16
    hc = H * C
    mesh = plsc.VectorSubcoreMesh(core_axis_name="c", subcore_axis_name="s")

    @functools.partial(
        pl.kernel,
        out_type=[
            jax.ShapeDtypeStruct((n_pad, C), jnp.float32),
            jax.ShapeDtypeStruct((n_pad, C), jnp.float32),
        ],
        mesh=mesh,
        compiler_params=pltpu.CompilerParams(use_tc_tiling_on_sc=False),
        scratch_types=[
            pltpu.VMEM((ch,), jnp.int32),
            pltpu.VMEM((ch,), jnp.int32),
            pltpu.VMEM((ch, hc), jnp.float32),
            pltpu.VMEM((ch, L), jnp.float32),
            pltpu.VMEM((ch, L), jnp.float32),
            pltpu.VMEM((ch, L), jnp.float32),
            pltpu.VMEM((ch, C), jnp.float32),
            pltpu.VMEM_SHARED((n_pad, C), jnp.float32),
            pltpu.SemaphoreType.DMA,
            pltpu.SemaphoreType.DMA,
            pltpu.SemaphoreType.DMA,
        ],
    )
    def kb(v_hbm, w_hbm, dp0, dp1, srci, dsti, acc0, acc1,
           sbuf, dbuf, vbuf, wbuf, d0buf, d1buf, ybuf, acc_sp,
           sem_v, sem_d0, sem_d1):
        cid = lax.axis_index("c")
        sid = lax.axis_index("s")
        wid = sid * 2 + cid

        zero16 = jnp.zeros((L,), jnp.float32)

        def zloop(i, carry):
            for j in range(C // L):
                ybuf[i, pl.ds(j * L, L)] = zero16
            return carry

        lax.fori_loop(0, ch, zloop, 0)

        def zcopy(i, carry):
            pltpu.sync_copy(ybuf, acc_sp.at[pl.ds(sid * npt + i * ch, ch)])
            return carry

        lax.fori_loop(0, npt // ch, zcopy, 0)
        plsc.subcore_barrier()

        ebase = wid * epw

        def chunk(g, carry):
            base = ebase + g * ch
            pltpu.sync_copy(srci.at[pl.ds(base, ch)], sbuf)
            pltpu.sync_copy(dsti.at[pl.ds(base, ch)], dbuf)
            cv = pltpu.async_copy(v_hbm.at[sbuf], vbuf, sem_v)
            c0 = pltpu.async_copy(dp0.at[dbuf], d0buf, sem_d0)
            c1 = pltpu.async_copy(dp1.at[dbuf], d1buf, sem_d1)
            pltpu.sync_copy(w_hbm.at[pl.ds(base, ch)], wbuf)
            cv.wait()
            c0.wait()
            c1.wait()

            def edge(e, ecarry):
                arow = wbuf[e] / (d0buf[e] + d1buf[e] + 1e-16)
                y0 = arow[0] * vbuf[e, pl.ds(0, L)]
                y1 = arow[0] * vbuf[e, pl.ds(L, L)]
                y2 = arow[0] * vbuf[e, pl.ds(2 * L, L)]
                y3 = arow[0] * vbuf[e, pl.ds(3 * L, L)]
                for h in range(1, H):
                    a_s = arow[h]
                    o = h * C
                    y0 = y0 + a_s * vbuf[e, pl.ds(o, L)]
                    y1 = y1 + a_s * vbuf[e, pl.ds(o + L, L)]
                    y2 = y2 + a_s * vbuf[e, pl.ds(o + 2 * L, L)]
                    y3 = y3 + a_s * vbuf[e, pl.ds(o + 3 * L, L)]
                ybuf[e, pl.ds(0, L)] = y0
                ybuf[e, pl.ds(L, L)] = y1
                ybuf[e, pl.ds(2 * L, L)] = y2
                ybuf[e, pl.ds(3 * L, L)] = y3
                return ecarry

            lax.fori_loop(0, ch, edge, 0)
            pltpu.sync_copy(ybuf, acc_sp.at[dbuf], add=True)
            return carry

        lax.fori_loop(0, nch, chunk, 0)
        plsc.subcore_barrier()

        nb = sid * npt

        @pl.when(cid == 0)
        def _():
            pltpu.sync_copy(acc_sp.at[pl.ds(nb, npt)], acc0.at[pl.ds(nb, npt)])

        @pl.when(cid == 1)
        def _():
            pltpu.sync_copy(acc_sp.at[pl.ds(nb, npt)], acc1.at[pl.ds(nb, npt)])

    return kb


def _pool_body(a0_ref, a1_ref, skip_ref, batch_ref, out_ref, psum, cnt):
    i = pl.program_id(0)
    nblk = pl.num_programs(0)

    @pl.when(i == 0)
    def _():
        psum[...] = jnp.zeros_like(psum)
        cnt[...] = jnp.zeros_like(cnt)

    outb = (a0_ref[...] + a1_ref[...]) * 0.125 + skip_ref[...]
    bb = batch_ref[0]
    blk = outb.shape[0]
    oh = (lax.broadcasted_iota(jnp.int32, (64, blk), 0) == bb).astype(jnp.float32)
    psum[...] += lax.dot_general(oh, outb, (((1,), (0,)), ((), ())),
                                 preferred_element_type=jnp.float32)
    cnt[...] += jnp.sum(oh, axis=1, keepdims=True)

    @pl.when(i == nblk - 1)
    def _():
        out_ref[...] = psum[...] / jnp.maximum(cnt[...], 1.0)


def _run_pool(a0, a1, skip, batch3d):
    n = a0.shape[0]
    nblk = batch3d.shape[0]
    blk = n // nblk
    return pl.pallas_call(
        _pool_body,
        grid=(nblk,),
        in_specs=[
            pl.BlockSpec((blk, C), lambda i: (i, 0)),
            pl.BlockSpec((blk, C), lambda i: (i, 0)),
            pl.BlockSpec((blk, C), lambda i: (i, 0)),
            pl.BlockSpec((1, 1, blk), lambda i: (i, 0, 0)),
        ],
        out_specs=pl.BlockSpec((64, C), lambda i: (0, 0)),
        out_shape=jax.ShapeDtypeStruct((64, C), jnp.float32),
        scratch_shapes=[
            pltpu.VMEM((64, C), jnp.float32),
            pltpu.VMEM((64, 1), jnp.float32),
        ],
    )(a0, a1, skip, batch3d)


_PERM = np.array([(l if c % 2 == 0 else 7 - l) * C + c
                  for c in range(C) for l in range(H)], dtype=np.int32)


def kernel(x, edge_index, batch, Wn, bn, Wq, bq, Wk, bk, Wv, bv, Ws, bs):
    n, d = x.shape
    e = edge_index.shape[1]
    n_pad = ((n + 1279) // 1280) * 1280

    src = edge_index[0].astype(jnp.int32)
    dst = edge_index[1].astype(jnp.int32)
    batch_p = jnp.concatenate(
        [batch.astype(jnp.int32), jnp.full((n_pad - n,), 64, jnp.int32)])
    batch3d = batch_p.reshape(10, 1, n_pad // 10)
    x_p = jnp.pad(x, ((0, n_pad - n), (0, 0)))

    wn_t = Wn.T
    wqp_t = Wq[_PERM].T
    bqp = bq[_PERM].reshape(1, -1)
    wkp_t = Wk[_PERM].T
    bkp = bk[_PERM].reshape(1, -1)
    wv_t = Wv.T
    bv2 = bv.reshape(1, -1)
    ws_t = Ws.T
    bs2 = bs.reshape(1, -1)

    qt, kt, v, skip = _run_proj(x_p, wn_t, bn.reshape(1, -1), wqp_t, bqp,
                                wkp_t, bkp, wv_t, bv2, ws_t, bs2)

    ka = _make_kernel_a(n_pad, e)
    w, dp0, dp1 = ka(qt, kt, src, dst)

    kb = _make_kernel_b(n_pad, e)
    acc0, acc1 = kb(v, w, dp0, dp1, src, dst)

    return _run_pool(acc0, acc1, skip, batch3d)


# speedup vs baseline: 18.5028x; 1.0209x over previous
import functools

import jax
import jax.numpy as jnp
import numpy as np
from jax import lax
from jax.experimental import pallas as pl
from jax.experimental.pallas import tpu as pltpu
from jax.experimental.pallas import tpu_sc as plsc

H = 8
C = 64
L = 16
NW = 32


def _proj_body(x_ref, wn, bn, wqp, bqp, wkp, bkp, wv, bv, ws, bs,
               qt_ref, kt_ref, v_ref, skip_ref):
    xb = x_ref[...]
    h = jnp.dot(xb, wn[...], preferred_element_type=jnp.float32) + bn[...]
    qt_ref[...] = (jnp.dot(h, wqp[...], preferred_element_type=jnp.float32)
                   + bqp[...]).astype(jnp.bfloat16)
    kt_ref[...] = (jnp.dot(h, wkp[...], preferred_element_type=jnp.float32)
                   + bkp[...]).astype(jnp.bfloat16)
    v_ref[...] = (jnp.dot(h, wv[...], preferred_element_type=jnp.float32)
                  + bv[...]).astype(jnp.bfloat16)
    skip_ref[...] = jnp.dot(h, ws[...], preferred_element_type=jnp.float32) + bs[...]


def _run_proj(x, wn, bn, wqp, bqp, wkp, bkp, wv, bv, ws, bs):
    n, d = x.shape
    hc = wqp.shape[1]
    blk = 1000
    grid = n // blk
    full = lambda r, c: pl.BlockSpec((r, c), lambda i: (0, 0))
    return pl.pallas_call(
        _proj_body,
        grid=(grid,),
        in_specs=[
            pl.BlockSpec((blk, d), lambda i: (i, 0)),
            full(d, d), full(1, d),
            full(d, hc), full(1, hc),
            full(d, hc), full(1, hc),
            full(d, hc), full(1, hc),
            full(d, C), full(1, C),
        ],
        out_specs=[
            pl.BlockSpec((blk, hc), lambda i: (i, 0)),
            pl.BlockSpec((blk, hc), lambda i: (i, 0)),
            pl.BlockSpec((blk, hc), lambda i: (i, 0)),
            pl.BlockSpec((blk, C), lambda i: (i, 0)),
        ],
        out_shape=[
            jax.ShapeDtypeStruct((n, hc), jnp.bfloat16),
            jax.ShapeDtypeStruct((n, hc), jnp.bfloat16),
            jax.ShapeDtypeStruct((n, hc), jnp.bfloat16),
            jax.ShapeDtypeStruct((n, C), jnp.float32),
        ],
    )(x, wn, bn, wqp, bqp, wkp, bkp, wv, bv, ws, bs)


def _make_kernel_a(n_pad, n_edges):
    ch = 40
    epw = n_edges // NW
    nch = epw // ch
    npt = n_pad // 16
    hc = H * C
    mesh = plsc.VectorSubcoreMesh(core_axis_name="c", subcore_axis_name="s")

    @functools.partial(
        pl.kernel,
        out_type=[
            jax.ShapeDtypeStruct((n_edges, L), jnp.float32),
            jax.ShapeDtypeStruct((n_pad, L), jnp.float32),
            jax.ShapeDtypeStruct((n_pad, L), jnp.float32),
        ],
        mesh=mesh,
        compiler_params=pltpu.CompilerParams(use_tc_tiling_on_sc=False, needs_layout_passes=False),
        scratch_types=[
            pltpu.VMEM((ch,), jnp.int32),
            pltpu.VMEM((ch,), jnp.int32),
            pltpu.VMEM((ch, hc // 2), jnp.int32),
            pltpu.VMEM((ch, hc // 2), jnp.int32),
            pltpu.VMEM((ch, L), jnp.float32),
            pltpu.VMEM_SHARED((n_pad, L), jnp.float32),
            pltpu.SemaphoreType.DMA,
            pltpu.SemaphoreType.DMA,
        ],
    )
    def ka(qt, kt, srci, dsti, w_out, dp0, dp1,
           sbuf, dbuf, qbuf, kbuf, wbuf, denom_sp, sem_q, sem_k):
        cid = lax.axis_index("c")
        sid = lax.axis_index("s")
        wid = sid * 2 + cid

        zero16 = jnp.zeros((L,), jnp.float32)

        def zloop(i, carry):
            wbuf[i] = zero16
            return carry

        lax.fori_loop(0, ch, zloop, 0)

        def zcopy(i, carry):
            pltpu.sync_copy(wbuf, denom_sp.at[pl.ds(sid * npt + i * ch, ch)])
            return carry

        lax.fori_loop(0, npt // ch, zcopy, 0)
        plsc.subcore_barrier()

        ebase = wid * epw

        def chunk(g, carry):
            base = ebase + g * ch
            pltpu.sync_copy(srci.at[pl.ds(base, ch)], sbuf)
            pltpu.sync_copy(dsti.at[pl.ds(base, ch)], dbuf)
            cq = pltpu.async_copy(qt.at[dbuf], qbuf, sem_q)
            ck = pltpu.async_copy(kt.at[sbuf], kbuf, sem_k)
            cq.wait()
            ck.wait()

            hi_mask = jnp.full((L,), -65536, jnp.int32)

            def bf2(x):
                lo = plsc.bitcast(lax.shift_left(x, 16), jnp.float32)
                hi = plsc.bitcast(lax.bitwise_and(x, hi_mask), jnp.float32)
                return lo, hi

            def edge(e, ecarry):
                accs = [None] * 4
                for i in range(16):
                    ql, qh = bf2(qbuf[e, pl.ds(i * L, L)])
                    kl, kh = bf2(kbuf[e, pl.ds(i * L, L)])
                    t = ql * kl + qh * kh
                    s = i % 4
                    accs[s] = t if accs[s] is None else accs[s] + t
                acc = (accs[0] + accs[1]) + (accs[2] + accs[3])
                logits = (acc + lax.rev(acc, (0,))) * 0.125
                wbuf[e] = jnp.exp(logits)
                return ecarry

            lax.fori_loop(0, ch, edge, 0)
            pltpu.sync_copy(wbuf, w_out.at[pl.ds(base, ch)])
            pltpu.sync_copy(wbuf, denom_sp.at[dbuf], add=True)
            return carry

        lax.fori_loop(0, nch, chunk, 0)
        plsc.subcore_barrier()

        nb = sid * npt

        @pl.when(cid == 0)
        def _():
            pltpu.sync_copy(denom_sp.at[pl.ds(nb, npt)], dp0.at[pl.ds(nb, npt)])

        @pl.when(cid == 1)
        def _():
            pltpu.sync_copy(denom_sp.at[pl.ds(nb, npt)], dp1.at[pl.ds(nb, npt)])

    return ka


def _make_kernel_b(n_pad, n_edges):
    ch = 40
    epw = n_edges // NW
    nch = epw // ch
    npt = n_pad // 16
    hc = H * C
    mesh = plsc.VectorSubcoreMesh(core_axis_name="c", subcore_axis_name="s")

    @functools.partial(
        pl.kernel,
        out_type=[
            jax.ShapeDtypeStruct((n_pad, C), jnp.float32),
            jax.ShapeDtypeStruct((n_pad, C), jnp.float32),
        ],
        mesh=mesh,
        compiler_params=pltpu.CompilerParams(use_tc_tiling_on_sc=False, needs_layout_passes=False),
        scratch_types=[
            pltpu.VMEM((ch,), jnp.int32),
            pltpu.VMEM((ch,), jnp.int32),
            pltpu.VMEM((ch, hc // 2), jnp.int32),
            pltpu.VMEM((ch, L), jnp.float32),
            pltpu.VMEM((ch, L), jnp.float32),
            pltpu.VMEM((ch, L), jnp.float32),
            pltpu.VMEM((ch, C), jnp.float32),
            pltpu.VMEM_SHARED((n_pad, C), jnp.float32),
            pltpu.SemaphoreType.DMA,
            pltpu.SemaphoreType.DMA,
            pltpu.SemaphoreType.DMA,
        ],
    )
    def kb(v_hbm, w_hbm, dp0, dp1, srci, dsti, acc0, acc1,
           sbuf, dbuf, vbuf, wbuf, d0buf, d1buf, ybuf, acc_sp,
           sem_v, sem_d0, sem_d1):
        cid = lax.axis_index("c")
        sid = lax.axis_index("s")
        wid = sid * 2 + cid

        zero16 = jnp.zeros((L,), jnp.float32)

        def zloop(i, carry):
            for j in range(C // L):
                ybuf[i, pl.ds(j * L, L)] = zero16
            return carry

        lax.fori_loop(0, ch, zloop, 0)

        def zcopy(i, carry):
            pltpu.sync_copy(ybuf, acc_sp.at[pl.ds(sid * npt + i * ch, ch)])
            return carry

        lax.fori_loop(0, npt // ch, zcopy, 0)
        plsc.subcore_barrier()

        ebase = wid * epw

        def chunk(g, carry):
            base = ebase + g * ch
            pltpu.sync_copy(srci.at[pl.ds(base, ch)], sbuf)
            pltpu.sync_copy(dsti.at[pl.ds(base, ch)], dbuf)
            cv = pltpu.async_copy(v_hbm.at[sbuf], vbuf, sem_v)
            c0 = pltpu.async_copy(dp0.at[dbuf], d0buf, sem_d0)
            c1 = pltpu.async_copy(dp1.at[dbuf], d1buf, sem_d1)
            pltpu.sync_copy(w_hbm.at[pl.ds(base, ch)], wbuf)
            cv.wait()
            c0.wait()
            c1.wait()

            hi_mask = jnp.full((L,), -65536, jnp.int32)

            def bf2(x):
                lo = plsc.bitcast(lax.shift_left(x, 16), jnp.float32)
                hi = plsc.bitcast(lax.bitwise_and(x, hi_mask), jnp.float32)
                return lo, hi

            def edge(e, ecarry):
                arow = wbuf[e] / (d0buf[e] + d1buf[e] + 1e-16)
                ys = [None] * 4
                for h in range(H):
                    a_s = arow[h]
                    o = h * (C // 2)
                    for half in range(2):
                        x0, x1 = bf2(vbuf[e, pl.ds(o + half * L, L)])
                        s0, s1 = 2 * half, 2 * half + 1
                        t0, t1 = a_s * x0, a_s * x1
                        ys[s0] = t0 if ys[s0] is None else ys[s0] + t0
                        ys[s1] = t1 if ys[s1] is None else ys[s1] + t1
                ybuf[e, pl.ds(0, L)] = ys[0]
                ybuf[e, pl.ds(L, L)] = ys[1]
                ybuf[e, pl.ds(2 * L, L)] = ys[2]
                ybuf[e, pl.ds(3 * L, L)] = ys[3]
                return ecarry

            lax.fori_loop(0, ch, edge, 0)
            pltpu.sync_copy(ybuf, acc_sp.at[dbuf], add=True)
            return carry

        lax.fori_loop(0, nch, chunk, 0)
        plsc.subcore_barrier()

        nb = sid * npt

        @pl.when(cid == 0)
        def _():
            pltpu.sync_copy(acc_sp.at[pl.ds(nb, npt)], acc0.at[pl.ds(nb, npt)])

        @pl.when(cid == 1)
        def _():
            pltpu.sync_copy(acc_sp.at[pl.ds(nb, npt)], acc1.at[pl.ds(nb, npt)])

    return kb


def _pool_body(a0_ref, a1_ref, skip_ref, batch_ref, out_ref, psum, cnt):
    i = pl.program_id(0)
    nblk = pl.num_programs(0)

    @pl.when(i == 0)
    def _():
        psum[...] = jnp.zeros_like(psum)
        cnt[...] = jnp.zeros_like(cnt)

    outb = (a0_ref[...] + a1_ref[...]) * 0.125 + skip_ref[...]
    bb = batch_ref[0]
    blk = outb.shape[0]
    oh = (lax.broadcasted_iota(jnp.int32, (64, blk), 0) == bb).astype(jnp.float32)
    psum[...] += lax.dot_general(oh, outb, (((1,), (0,)), ((), ())),
                                 preferred_element_type=jnp.float32)
    cnt[...] += jnp.sum(oh, axis=1, keepdims=True)

    @pl.when(i == nblk - 1)
    def _():
        out_ref[...] = psum[...] / jnp.maximum(cnt[...], 1.0)


def _run_pool(a0, a1, skip, batch3d):
    n = a0.shape[0]
    nblk = batch3d.shape[0]
    blk = n // nblk
    return pl.pallas_call(
        _pool_body,
        grid=(nblk,),
        in_specs=[
            pl.BlockSpec((blk, C), lambda i: (i, 0)),
            pl.BlockSpec((blk, C), lambda i: (i, 0)),
            pl.BlockSpec((blk, C), lambda i: (i, 0)),
            pl.BlockSpec((1, 1, blk), lambda i: (i, 0, 0)),
        ],
        out_specs=pl.BlockSpec((64, C), lambda i: (0, 0)),
        out_shape=jax.ShapeDtypeStruct((64, C), jnp.float32),
        scratch_shapes=[
            pltpu.VMEM((64, C), jnp.float32),
            pltpu.VMEM((64, 1), jnp.float32),
        ],
    )(a0, a1, skip, batch3d)


def _qk_perm():
    p = np.arange(H * C)
    i, r = p // 32, p % 32
    l, j = r // 2, r % 2
    h = np.where(l < 8, l, 15 - l)
    c = np.where(l < 8, 2 * i + j, 32 + 2 * i + j)
    return (h * C + c).astype(np.int32)


def _v_perm():
    p = np.arange(H * C)
    h, r = p // 64, p % 64
    half, r2 = r // 32, r % 32
    l, j = r2 // 2, r2 % 2
    c = 32 * half + 16 * j + l
    return (h * C + c).astype(np.int32)


_PERM = _qk_perm()
_PERMV = _v_perm()


def kernel(x, edge_index, batch, Wn, bn, Wq, bq, Wk, bk, Wv, bv, Ws, bs):
    n, d = x.shape
    e = edge_index.shape[1]
    n_pad = ((n + 1279) // 1280) * 1280

    src = edge_index[0].astype(jnp.int32)
    dst = edge_index[1].astype(jnp.int32)
    batch_p = jnp.concatenate(
        [batch.astype(jnp.int32), jnp.full((n_pad - n,), 64, jnp.int32)])
    batch3d = batch_p.reshape(10, 1, n_pad // 10)
    x_p = jnp.pad(x, ((0, n_pad - n), (0, 0)))

    wn_t = Wn.T
    wqp_t = Wq[_PERM].T
    bqp = bq[_PERM].reshape(1, -1)
    wkp_t = Wk[_PERM].T
    bkp = bk[_PERM].reshape(1, -1)
    wv_t = Wv[_PERMV].T
    bv2 = bv[_PERMV].reshape(1, -1)
    ws_t = Ws.T
    bs2 = bs.reshape(1, -1)

    qt, kt, v, skip = _run_proj(x_p, wn_t, bn.reshape(1, -1), wqp_t, bqp,
                                wkp_t, bkp, wv_t, bv2, ws_t, bs2)

    def as_i32(a):
        m = a.shape[1] // 2
        return lax.bitcast_convert_type(a.reshape(n_pad, m, 2), jnp.int32)

    ka = _make_kernel_a(n_pad, e)
    w, dp0, dp1 = ka(as_i32(qt), as_i32(kt), src, dst)

    kb = _make_kernel_b(n_pad, e)
    acc0, acc1 = kb(as_i32(v), w, dp0, dp1, src, dst)

    return _run_pool(acc0, acc1, skip, batch3d)
